# baseline (speedup 1.0000x reference)
import functools

import jax
import jax.numpy as jnp
from jax import lax
from jax.experimental import pallas as pl
from jax.experimental.pallas import tpu as pltpu
from jax.experimental.pallas import tpu_sc as plsc

N_NODES = 10000
N_EDGES = 320000
D = 128
EA = 16
NC = 2
NS = 16
NW = NC * NS
CHUNK = 128
NCHUNK = 80
SCH = 8
NSUPER = NCHUNK // SCH
EDGES_PER_TILE = NCHUNK * CHUNK
NE_PAD = NW * EDGES_PER_TILE
ACC_ROWS = 10112
ROWS_PER_TILE = ACC_ROWS // NS


def _sc_mesh():
    return plsc.VectorSubcoreMesh(
        core_axis_name="c", subcore_axis_name="s", num_cores=NC, num_subcores=NS
    )


_ZF = ROWS_PER_TILE // CHUNK
_ZR = ROWS_PER_TILE % CHUNK


CH_AG = 64
NCH_AG = EDGES_PER_TILE // CH_AG


def _make_agg():
    out_type = [jax.ShapeDtypeStruct((NC, ACC_ROWS, D), jnp.float32)]
    scratch = [
        pltpu.VMEM((16, CH_AG), jnp.int32),
        pltpu.VMEM((16, CH_AG), jnp.int32),
        pltpu.VMEM((4, CH_AG, D), jnp.float32),
        pltpu.VMEM_SHARED((ACC_ROWS, D), jnp.float32),
    ] + [pltpu.SemaphoreType.DMA] * 8

    def body(g, src3d, dst3d, pacc,
             sidx_v, didx_v, ring, acc_sh, *sems):
        sg = sems[:4]
        ss = sems[4:]
        c = lax.axis_index("c")
        s = lax.axis_index("s")
        wid = s * NC + c
        base = s * ROWS_PER_TILE
        bufs = [ring.at[b] for b in range(4)]
        zbuf = ring.at[0]
        zv = jnp.zeros((16,), jnp.float32)

        def fill(i, carry):
            for k in range(D // 16):
                ring[0, i, pl.ds(k * 16, 16)] = zv
            return carry

        lax.fori_loop(0, CH_AG, fill, 0)
        for k in range(ROWS_PER_TILE // CH_AG):
            pltpu.sync_copy(zbuf, acc_sh.at[pl.ds(base + k * CH_AG, CH_AG)])
        pltpu.sync_copy(zbuf.at[pl.ds(0, ROWS_PER_TILE % CH_AG)],
                        acc_sh.at[pl.ds(base + (ROWS_PER_TILE // CH_AG) * CH_AG,
                                        ROWS_PER_TILE % CH_AG)])
        plsc.subcore_barrier()

        def group(grp, carry):
            bj = grp * 16
            pltpu.sync_copy(src3d.at[wid, pl.ds(bj, 16)], sidx_v)
            pltpu.sync_copy(dst3d.at[wid, pl.ds(bj, 16)], didx_v)
            dg = [None] * 4
            ds = [None] * 4
            for q in range(16):
                b = q % 4
                if q >= 4:
                    ds[b].wait()
                dg[b] = pltpu.async_copy(g.at[sidx_v.at[q]], bufs[b],
                                         sg[b])
                if q >= 2:
                    qq = q - 2
                    bb = qq % 4
                    dg[bb].wait()
                    ds[bb] = pltpu.async_copy(bufs[bb],
                                              acc_sh.at[didx_v.at[qq]],
                                              ss[bb], add=True)
            for qq in (14, 15):
                bb = qq % 4
                dg[bb].wait()
                ds[bb] = pltpu.async_copy(bufs[bb], acc_sh.at[didx_v.at[qq]],
                                          ss[bb], add=True)
            for bb in range(4):
                ds[bb].wait()
            return carry

        lax.fori_loop(0, NCH_AG // 16, group, 0)
        plsc.subcore_barrier()
        nf = ROWS_PER_TILE // CH_AG
        nr = ROWS_PER_TILE % CH_AG
        for k in range(nf):
            pltpu.sync_copy(acc_sh.at[pl.ds(base + k * CH_AG, CH_AG)], zbuf)
            pltpu.sync_copy(zbuf, pacc.at[c, pl.ds(base + k * CH_AG, CH_AG)])
        pltpu.sync_copy(acc_sh.at[pl.ds(base + nf * CH_AG, nr)],
                        zbuf.at[pl.ds(0, nr)])
        pltpu.sync_copy(zbuf.at[pl.ds(0, nr)],
                        pacc.at[c, pl.ds(base + nf * CH_AG, nr)])

    return pl.kernel(body, out_type=out_type, mesh=_sc_mesh(),
                     scratch_types=scratch)


def _make_cnt():
    out_type = [jax.ShapeDtypeStruct((NC, ACC_ROWS, D), jnp.float32)]
    scratch = [
        pltpu.VMEM((NCHUNK, CHUNK), jnp.int32),
        pltpu.VMEM((CHUNK, D), jnp.float32),
        pltpu.VMEM((CHUNK, D), jnp.float32),
        pltpu.VMEM_SHARED((ACC_ROWS, D), jnp.float32),
    ]

    def body(dst3d, pcnt, dst_v, ones_v, st_v, cnt_sh):
        c = lax.axis_index("c")
        s = lax.axis_index("s")
        wid = s * NC + c
        base = s * ROWS_PER_TILE
        zv = jnp.zeros((16,), jnp.float32)
        ov = jnp.ones((16,), jnp.float32)

        def fill(i, carry):
            for k in range(D // 16):
                ones_v[i, pl.ds(k * 16, 16)] = ov
                st_v[i, pl.ds(k * 16, 16)] = zv
            return carry

        lax.fori_loop(0, CHUNK, fill, 0)
        for k in range(_ZF):
            pltpu.sync_copy(st_v, cnt_sh.at[pl.ds(base + k * CHUNK, CHUNK)])
        pltpu.sync_copy(st_v.at[pl.ds(0, _ZR)],
                        cnt_sh.at[pl.ds(base + _ZF * CHUNK, _ZR)])
        pltpu.sync_copy(dst3d.at[wid], dst_v)
        plsc.subcore_barrier()

        def step(j, carry):
            pltpu.sync_copy(ones_v, cnt_sh.at[dst_v.at[j]], add=True)
            return carry

        lax.fori_loop(0, NCHUNK, step, 0)
        plsc.subcore_barrier()
        for k in range(_ZF):
            pltpu.sync_copy(cnt_sh.at[pl.ds(base + k * CHUNK, CHUNK)], st_v)
            pltpu.sync_copy(st_v, pcnt.at[c, pl.ds(base + k * CHUNK, CHUNK)])
        pltpu.sync_copy(cnt_sh.at[pl.ds(base + _ZF * CHUNK, _ZR)],
                        st_v.at[pl.ds(0, _ZR)])
        pltpu.sync_copy(st_v.at[pl.ds(0, _ZR)],
                        pcnt.at[c, pl.ds(base + _ZF * CHUNK, _ZR)])

    return pl.kernel(body, out_type=out_type, mesh=_sc_mesh(),
                     scratch_types=scratch)


@functools.lru_cache(maxsize=None)
def _get_agg():
    return _make_agg()


@functools.lru_cache(maxsize=None)
def _get_cnt():
    return _make_cnt()


def _cnt(dst3d):
    (pc,) = _get_cnt()(dst3d)
    return pc


def _agg(g, src2d, dst2d):
    (p,) = _get_agg()(g, src2d, dst2d)
    return p


def _make_gather2():
    out_type = [
        jax.ShapeDtypeStruct((NE_PAD, D), jnp.float32),
        jax.ShapeDtypeStruct((NE_PAD, D), jnp.float32),
    ]
    scratch = [
        pltpu.VMEM((NCHUNK, CHUNK), jnp.int32),
        pltpu.VMEM((4, CHUNK, D), jnp.float32),
    ] + [pltpu.SemaphoreType.DMA] * 8

    def body(g, src3d, dst3d, hi, hj, idx_v, ring, *sems):
        sg = sems[:4]
        sw = sems[4:]
        c = lax.axis_index("c")
        s = lax.axis_index("s")
        bufs = [ring.at[b] for b in range(4)]

        def pipe(idx3d, out):
            for ph in range(2):
                r = 2 * s + ph
                pltpu.sync_copy(idx3d.at[r], idx_v)

                def group(grp, carry):
                    bj = grp * 16
                    base_e = r * EDGES_PER_TILE + bj * CHUNK
                    dg = [None] * 4
                    dw = [None] * 4
                    for q in range(16):
                        b = q % 4
                        if q >= 4:
                            dw[b].wait()
                        dg[b] = pltpu.async_copy(g.at[idx_v.at[bj + q]],
                                                 bufs[b], sg[b])
                        if q >= 2:
                            qq = q - 2
                            bb = qq % 4
                            dg[bb].wait()
                            dw[bb] = pltpu.async_copy(
                                bufs[bb],
                                out.at[pl.ds(base_e + qq * CHUNK, CHUNK)],
                                sw[bb])
                    for qq in (14, 15):
                        bb = qq % 4
                        dg[bb].wait()
                        dw[bb] = pltpu.async_copy(
                            bufs[bb],
                            out.at[pl.ds(base_e + qq * CHUNK, CHUNK)],
                            sw[bb])
                    for bb in range(4):
                        dw[bb].wait()
                    return carry

                lax.fori_loop(0, NCHUNK // 16, group, 0)

        pl.when(c == 0)(lambda: pipe(src3d, hi))
        pl.when(c == 1)(lambda: pipe(dst3d, hj))

    return pl.kernel(body, out_type=out_type, mesh=_sc_mesh(),
                     scratch_types=scratch)


@functools.lru_cache(maxsize=None)
def _get_gather2():
    return _make_gather2()


def _gather2(g, src2d, dst2d):
    return _get_gather2()(g, src2d, dst2d)


_RB = 1000
_RBE = 1280


def _gru_tc(x, h, wihT, whhT, bih, bhh):
    def body(x_r, h_r, wi_r, wh_r, bi_r, bh_r, o_r):
        hb = h_r[...]
        gi = jnp.dot(x_r[...], wi_r[...], preferred_element_type=jnp.float32) + bi_r[...]
        gh = jnp.dot(hb, wh_r[...], preferred_element_type=jnp.float32) + bh_r[...]
        r = jax.nn.sigmoid(gi[:, :D] + gh[:, :D])
        z = jax.nn.sigmoid(gi[:, D:2 * D] + gh[:, D:2 * D])
        n = jnp.tanh(gi[:, 2 * D:] + r * gh[:, 2 * D:])
        o_r[...] = (1.0 - z) * n + z * hb

    return pl.pallas_call(
        body,
        grid=(N_NODES // _RB,),
        in_specs=[
            pl.BlockSpec((_RB, D), lambda i: (i, 0)),
            pl.BlockSpec((_RB, D), lambda i: (i, 0)),
            pl.BlockSpec((D, 3 * D), lambda i: (0, 0)),
            pl.BlockSpec((D, 3 * D), lambda i: (0, 0)),
            pl.BlockSpec((1, 3 * D), lambda i: (0, 0)),
            pl.BlockSpec((1, 3 * D), lambda i: (0, 0)),
        ],
        out_specs=pl.BlockSpec((_RB, D), lambda i: (i, 0)),
        out_shape=jax.ShapeDtypeStruct((N_NODES, D), jnp.float32),
    )(x, h, wihT, whhT, bih.reshape(1, -1), bhh.reshape(1, -1))


def _sage_tc(pacc, pcnt, g, wlT, wrT, bl):
    def body(p_r, c_r, g_r, wl_r, wr_r, bl_r, o_r):
        ssum = p_r[0] + p_r[1]
        cnt = c_r[0] + c_r[1]
        inv = 1.0 / jnp.maximum(cnt[:, 0:1], 1.0)
        mean = ssum * inv
        acc = jnp.dot(mean, wl_r[...], preferred_element_type=jnp.float32)
        acc = acc + jnp.dot(g_r[...], wr_r[...], preferred_element_type=jnp.float32)
        o_r[...] = jnp.maximum(acc + bl_r[...], 0.0)

    return pl.pallas_call(
        body,
        grid=(N_NODES // _RB,),
        in_specs=[
            pl.BlockSpec((NC, _RB, D), lambda i: (0, i, 0)),
            pl.BlockSpec((NC, _RB, D), lambda i: (0, i, 0)),
            pl.BlockSpec((_RB, D), lambda i: (i, 0)),
            pl.BlockSpec((D, D), lambda i: (0, 0)),
            pl.BlockSpec((D, D), lambda i: (0, 0)),
            pl.BlockSpec((1, D), lambda i: (0, 0)),
        ],
        out_specs=pl.BlockSpec((_RB, D), lambda i: (i, 0)),
        out_shape=jax.ShapeDtypeStruct((N_NODES, D), jnp.float32),
    )(pacc, pcnt, g, wlT, wrT, bl.reshape(1, -1))


def _cls_tc(hi, hj, ea, w1aT, w1bT, w1cT, b1, w2, b2):
    def body(hi_r, hj_r, ea_r, wa_r, wb_r, wc_r, b1_r, w2_r, b2_r, o_r):
        hid = jnp.dot(hi_r[...], wa_r[...], preferred_element_type=jnp.float32)
        hid = hid + jnp.dot(hj_r[...], wb_r[...], preferred_element_type=jnp.float32)
        hid = hid + jnp.dot(ea_r[...], wc_r[...], preferred_element_type=jnp.float32)
        hid = jnp.maximum(hid + b1_r[...], 0.0)
        o_r[...] = jnp.sum(hid * w2_r[...], axis=1, keepdims=True) + b2_r[...]

    return pl.pallas_call(
        body,
        grid=(N_EDGES // _RBE,),
        in_specs=[
            pl.BlockSpec((_RBE, D), lambda i: (i, 0)),
            pl.BlockSpec((_RBE, D), lambda i: (i, 0)),
            pl.BlockSpec((_RBE, EA), lambda i: (i, 0)),
            pl.BlockSpec((D, 2 * D), lambda i: (0, 0)),
            pl.BlockSpec((D, 2 * D), lambda i: (0, 0)),
            pl.BlockSpec((EA, 2 * D), lambda i: (0, 0)),
            pl.BlockSpec((1, 2 * D), lambda i: (0, 0)),
            pl.BlockSpec((1, 2 * D), lambda i: (0, 0)),
            pl.BlockSpec((1, 1), lambda i: (0, 0)),
        ],
        out_specs=pl.BlockSpec((_RBE, 1), lambda i: (i, 0)),
        out_shape=jax.ShapeDtypeStruct((N_EDGES, 1), jnp.float32),
    )(hi, hj, ea, w1aT, w1bT, w1cT, b1.reshape(1, -1), w2, b2.reshape(1, 1))


def kernel(x, edge_index, edge_attr, h,
           W_ih, W_hh, b_ih, b_hh,
           Wl1, bl1, Wr1, Wl2, bl2, Wr2, Wl3, bl3, Wr3,
           Wc1, bc1, Wc2, bc2):
    src = edge_index[0].astype(jnp.int32)
    dst = edge_index[1].astype(jnp.int32)
    pad = NE_PAD - N_EDGES
    src_p = jnp.concatenate([src, jnp.zeros((pad,), jnp.int32)])
    dst_p = jnp.concatenate([dst, jnp.full((pad,), N_NODES, jnp.int32)])
    src2d = src_p.reshape(NW, NCHUNK, CHUNK)
    dst2d = dst_p.reshape(NW, NCHUNK, CHUNK)
    src64 = src_p.reshape(NW, NCH_AG, CH_AG)
    dst64 = dst_p.reshape(NW, NCH_AG, CH_AG)
    pc = _cnt(dst2d)
    h1 = _gru_tc(x, h, W_ih.T, W_hh.T, b_ih, b_hh)
    p1 = _agg(h1, src64, dst64)
    g1 = _sage_tc(p1, pc, h1, Wl1.T, Wr1.T, bl1)
    p2 = _agg(g1, src64, dst64)
    g2 = _sage_tc(p2, pc, g1, Wl2.T, Wr2.T, bl2)
    p3 = _agg(g2, src64, dst64)
    g3 = _sage_tc(p3, pc, g2, Wl3.T, Wr3.T, bl3)
    hi, hj = _gather2(g3, src2d, dst2d)
    out = _cls_tc(hi, hj, edge_attr,
                  Wc1[:, :D].T, Wc1[:, D:2 * D].T, Wc1[:, 2 * D:].T,
                  bc1, Wc2, bc2)
    return (out, g3)


# speedup vs baseline: 1.0240x; 1.0240x over previous
import functools

import jax
import jax.numpy as jnp
from jax import lax
from jax.experimental import pallas as pl
from jax.experimental.pallas import tpu as pltpu
from jax.experimental.pallas import tpu_sc as plsc

N_NODES = 10000
N_EDGES = 320000
D = 128
EA = 16
NC = 2
NS = 16
NW = NC * NS
CHUNK = 128
NCHUNK = 80
SCH = 8
NSUPER = NCHUNK // SCH
EDGES_PER_TILE = NCHUNK * CHUNK
NE_PAD = NW * EDGES_PER_TILE
ACC_ROWS = 10112
ROWS_PER_TILE = ACC_ROWS // NS


def _sc_mesh():
    return plsc.VectorSubcoreMesh(
        core_axis_name="c", subcore_axis_name="s", num_cores=NC, num_subcores=NS
    )


_ZF = ROWS_PER_TILE // CHUNK
_ZR = ROWS_PER_TILE % CHUNK


CH_AG = 64
NCH_AG = EDGES_PER_TILE // CH_AG


def _make_agg():
    out_type = [jax.ShapeDtypeStruct((NC, ACC_ROWS, D), jnp.float32)]
    scratch = [
        pltpu.VMEM((16, CH_AG), jnp.int32),
        pltpu.VMEM((16, CH_AG), jnp.int32),
        pltpu.VMEM((CH_AG, D), jnp.float32),
        pltpu.VMEM((CH_AG, D), jnp.float32),
        pltpu.VMEM((CH_AG, D), jnp.float32),
        pltpu.VMEM((CH_AG, D), jnp.float32),
        pltpu.VMEM_SHARED((ACC_ROWS, D), jnp.float32),
    ] + [pltpu.SemaphoreType.DMA] * 8

    def body(g, src3d, dst3d, pacc,
             sidx_v, didx_v, buf0, buf1, buf2, buf3, acc_sh, *sems):
        sg = sems[:4]
        ss = sems[4:]
        c = lax.axis_index("c")
        s = lax.axis_index("s")
        wid = s * NC + c
        base = s * ROWS_PER_TILE
        bufs = [buf0, buf1, buf2, buf3]
        zbuf = buf0
        zv = jnp.zeros((16,), jnp.float32)

        def fill(i, carry):
            for k in range(D // 16):
                buf0[i, pl.ds(k * 16, 16)] = zv
            return carry

        lax.fori_loop(0, CH_AG, fill, 0)
        for k in range(ROWS_PER_TILE // CH_AG):
            pltpu.sync_copy(zbuf, acc_sh.at[pl.ds(base + k * CH_AG, CH_AG)])
        pltpu.sync_copy(zbuf.at[pl.ds(0, ROWS_PER_TILE % CH_AG)],
                        acc_sh.at[pl.ds(base + (ROWS_PER_TILE // CH_AG) * CH_AG,
                                        ROWS_PER_TILE % CH_AG)])
        plsc.subcore_barrier()

        def group(grp, carry):
            bj = grp * 16
            pltpu.sync_copy(src3d.at[wid, pl.ds(bj, 16)], sidx_v)
            pltpu.sync_copy(dst3d.at[wid, pl.ds(bj, 16)], didx_v)
            dg = [None] * 4
            ds = [None] * 4
            for q in range(16):
                b = q % 4
                if q >= 4:
                    ds[b].wait()
                dg[b] = pltpu.async_copy(g.at[sidx_v.at[q]], bufs[b],
                                         sg[b])
                if q >= 2:
                    qq = q - 2
                    bb = qq % 4
                    dg[bb].wait()
                    ds[bb] = pltpu.async_copy(bufs[bb],
                                              acc_sh.at[didx_v.at[qq]],
                                              ss[bb], add=True)
            for qq in (14, 15):
                bb = qq % 4
                dg[bb].wait()
                ds[bb] = pltpu.async_copy(bufs[bb], acc_sh.at[didx_v.at[qq]],
                                          ss[bb], add=True)
            for bb in range(4):
                ds[bb].wait()
            return carry

        lax.fori_loop(0, NCH_AG // 16, group, 0)
        plsc.subcore_barrier()
        nf = ROWS_PER_TILE // CH_AG
        nr = ROWS_PER_TILE % CH_AG
        for k in range(nf):
            pltpu.sync_copy(acc_sh.at[pl.ds(base + k * CH_AG, CH_AG)], zbuf)
            pltpu.sync_copy(zbuf, pacc.at[c, pl.ds(base + k * CH_AG, CH_AG)])
        pltpu.sync_copy(acc_sh.at[pl.ds(base + nf * CH_AG, nr)],
                        zbuf.at[pl.ds(0, nr)])
        pltpu.sync_copy(zbuf.at[pl.ds(0, nr)],
                        pacc.at[c, pl.ds(base + nf * CH_AG, nr)])

    return pl.kernel(body, out_type=out_type, mesh=_sc_mesh(),
                     scratch_types=scratch)


def _make_cnt():
    out_type = [jax.ShapeDtypeStruct((NC, ACC_ROWS, D), jnp.float32)]
    scratch = [
        pltpu.VMEM((NCHUNK, CHUNK), jnp.int32),
        pltpu.VMEM((CHUNK, D), jnp.float32),
        pltpu.VMEM((CHUNK, D), jnp.float32),
        pltpu.VMEM_SHARED((ACC_ROWS, D), jnp.float32),
    ]

    def body(dst3d, pcnt, dst_v, ones_v, st_v, cnt_sh):
        c = lax.axis_index("c")
        s = lax.axis_index("s")
        wid = s * NC + c
        base = s * ROWS_PER_TILE
        zv = jnp.zeros((16,), jnp.float32)
        ov = jnp.ones((16,), jnp.float32)

        def fill(i, carry):
            for k in range(D // 16):
                ones_v[i, pl.ds(k * 16, 16)] = ov
                st_v[i, pl.ds(k * 16, 16)] = zv
            return carry

        lax.fori_loop(0, CHUNK, fill, 0)
        for k in range(_ZF):
            pltpu.sync_copy(st_v, cnt_sh.at[pl.ds(base + k * CHUNK, CHUNK)])
        pltpu.sync_copy(st_v.at[pl.ds(0, _ZR)],
                        cnt_sh.at[pl.ds(base + _ZF * CHUNK, _ZR)])
        pltpu.sync_copy(dst3d.at[wid], dst_v)
        plsc.subcore_barrier()

        def step(j, carry):
            pltpu.sync_copy(ones_v, cnt_sh.at[dst_v.at[j]], add=True)
            return carry

        lax.fori_loop(0, NCHUNK, step, 0)
        plsc.subcore_barrier()
        for k in range(_ZF):
            pltpu.sync_copy(cnt_sh.at[pl.ds(base + k * CHUNK, CHUNK)], st_v)
            pltpu.sync_copy(st_v, pcnt.at[c, pl.ds(base + k * CHUNK, CHUNK)])
        pltpu.sync_copy(cnt_sh.at[pl.ds(base + _ZF * CHUNK, _ZR)],
                        st_v.at[pl.ds(0, _ZR)])
        pltpu.sync_copy(st_v.at[pl.ds(0, _ZR)],
                        pcnt.at[c, pl.ds(base + _ZF * CHUNK, _ZR)])

    return pl.kernel(body, out_type=out_type, mesh=_sc_mesh(),
                     scratch_types=scratch)


@functools.lru_cache(maxsize=None)
def _get_agg():
    return _make_agg()


@functools.lru_cache(maxsize=None)
def _get_cnt():
    return _make_cnt()


def _cnt(dst3d):
    (pc,) = _get_cnt()(dst3d)
    return pc


def _agg(g, src2d, dst2d):
    (p,) = _get_agg()(g, src2d, dst2d)
    return p


def _make_gather2():
    out_type = [
        jax.ShapeDtypeStruct((NE_PAD, D), jnp.float32),
        jax.ShapeDtypeStruct((NE_PAD, D), jnp.float32),
    ]
    scratch = [
        pltpu.VMEM((NCHUNK, CHUNK), jnp.int32),
        pltpu.VMEM((CHUNK, D), jnp.float32),
        pltpu.VMEM((CHUNK, D), jnp.float32),
        pltpu.VMEM((CHUNK, D), jnp.float32),
        pltpu.VMEM((CHUNK, D), jnp.float32),
    ] + [pltpu.SemaphoreType.DMA] * 8

    def body(g, src3d, dst3d, hi, hj, idx_v, buf0, buf1, buf2, buf3, *sems):
        sg = sems[:4]
        sw = sems[4:]
        c = lax.axis_index("c")
        s = lax.axis_index("s")
        bufs = [buf0, buf1, buf2, buf3]

        def pipe(idx3d, out):
            for ph in range(2):
                r = 2 * s + ph
                pltpu.sync_copy(idx3d.at[r], idx_v)

                def group(grp, carry):
                    bj = grp * 16
                    base_e = r * EDGES_PER_TILE + bj * CHUNK
                    dg = [None] * 4
                    dw = [None] * 4
                    for q in range(16):
                        b = q % 4
                        if q >= 4:
                            dw[b].wait()
                        dg[b] = pltpu.async_copy(g.at[idx_v.at[bj + q]],
                                                 bufs[b], sg[b])
                        if q >= 2:
                            qq = q - 2
                            bb = qq % 4
                            dg[bb].wait()
                            dw[bb] = pltpu.async_copy(
                                bufs[bb],
                                out.at[pl.ds(base_e + qq * CHUNK, CHUNK)],
                                sw[bb])
                    for qq in (14, 15):
                        bb = qq % 4
                        dg[bb].wait()
                        dw[bb] = pltpu.async_copy(
                            bufs[bb],
                            out.at[pl.ds(base_e + qq * CHUNK, CHUNK)],
                            sw[bb])
                    for bb in range(4):
                        dw[bb].wait()
                    return carry

                lax.fori_loop(0, NCHUNK // 16, group, 0)

        pl.when(c == 0)(lambda: pipe(src3d, hi))
        pl.when(c == 1)(lambda: pipe(dst3d, hj))

    return pl.kernel(body, out_type=out_type, mesh=_sc_mesh(),
                     scratch_types=scratch)


@functools.lru_cache(maxsize=None)
def _get_gather2():
    return _make_gather2()


def _gather2(g, src2d, dst2d):
    return _get_gather2()(g, src2d, dst2d)


_RB = 1000
_RBE = 1280


def _gru_tc(x, h, wihT, whhT, bih, bhh):
    def body(x_r, h_r, wi_r, wh_r, bi_r, bh_r, o_r):
        hb = h_r[...]
        gi = jnp.dot(x_r[...], wi_r[...], preferred_element_type=jnp.float32) + bi_r[...]
        gh = jnp.dot(hb, wh_r[...], preferred_element_type=jnp.float32) + bh_r[...]
        r = jax.nn.sigmoid(gi[:, :D] + gh[:, :D])
        z = jax.nn.sigmoid(gi[:, D:2 * D] + gh[:, D:2 * D])
        n = jnp.tanh(gi[:, 2 * D:] + r * gh[:, 2 * D:])
        o_r[...] = (1.0 - z) * n + z * hb

    return pl.pallas_call(
        body,
        grid=(N_NODES // _RB,),
        in_specs=[
            pl.BlockSpec((_RB, D), lambda i: (i, 0)),
            pl.BlockSpec((_RB, D), lambda i: (i, 0)),
            pl.BlockSpec((D, 3 * D), lambda i: (0, 0)),
            pl.BlockSpec((D, 3 * D), lambda i: (0, 0)),
            pl.BlockSpec((1, 3 * D), lambda i: (0, 0)),
            pl.BlockSpec((1, 3 * D), lambda i: (0, 0)),
        ],
        out_specs=pl.BlockSpec((_RB, D), lambda i: (i, 0)),
        out_shape=jax.ShapeDtypeStruct((N_NODES, D), jnp.float32),
    )(x, h, wihT, whhT, bih.reshape(1, -1), bhh.reshape(1, -1))


def _sage_tc(pacc, pcnt, g, wlT, wrT, bl):
    def body(p_r, c_r, g_r, wl_r, wr_r, bl_r, o_r):
        ssum = p_r[0] + p_r[1]
        cnt = c_r[0] + c_r[1]
        inv = 1.0 / jnp.maximum(cnt[:, 0:1], 1.0)
        mean = ssum * inv
        acc = jnp.dot(mean, wl_r[...], preferred_element_type=jnp.float32)
        acc = acc + jnp.dot(g_r[...], wr_r[...], preferred_element_type=jnp.float32)
        o_r[...] = jnp.maximum(acc + bl_r[...], 0.0)

    return pl.pallas_call(
        body,
        grid=(N_NODES // _RB,),
        in_specs=[
            pl.BlockSpec((NC, _RB, D), lambda i: (0, i, 0)),
            pl.BlockSpec((NC, _RB, D), lambda i: (0, i, 0)),
            pl.BlockSpec((_RB, D), lambda i: (i, 0)),
            pl.BlockSpec((D, D), lambda i: (0, 0)),
            pl.BlockSpec((D, D), lambda i: (0, 0)),
            pl.BlockSpec((1, D), lambda i: (0, 0)),
        ],
        out_specs=pl.BlockSpec((_RB, D), lambda i: (i, 0)),
        out_shape=jax.ShapeDtypeStruct((N_NODES, D), jnp.float32),
    )(pacc, pcnt, g, wlT, wrT, bl.reshape(1, -1))


def _cls_tc(hi, hj, ea, w1aT, w1bT, w1cT, b1, w2, b2):
    def body(hi_r, hj_r, ea_r, wa_r, wb_r, wc_r, b1_r, w2_r, b2_r, o_r):
        hid = jnp.dot(hi_r[...], wa_r[...], preferred_element_type=jnp.float32)
        hid = hid + jnp.dot(hj_r[...], wb_r[...], preferred_element_type=jnp.float32)
        hid = hid + jnp.dot(ea_r[...], wc_r[...], preferred_element_type=jnp.float32)
        hid = jnp.maximum(hid + b1_r[...], 0.0)
        o_r[...] = jnp.sum(hid * w2_r[...], axis=1, keepdims=True) + b2_r[...]

    return pl.pallas_call(
        body,
        grid=(N_EDGES // _RBE,),
        in_specs=[
            pl.BlockSpec((_RBE, D), lambda i: (i, 0)),
            pl.BlockSpec((_RBE, D), lambda i: (i, 0)),
            pl.BlockSpec((_RBE, EA), lambda i: (i, 0)),
            pl.BlockSpec((D, 2 * D), lambda i: (0, 0)),
            pl.BlockSpec((D, 2 * D), lambda i: (0, 0)),
            pl.BlockSpec((EA, 2 * D), lambda i: (0, 0)),
            pl.BlockSpec((1, 2 * D), lambda i: (0, 0)),
            pl.BlockSpec((1, 2 * D), lambda i: (0, 0)),
            pl.BlockSpec((1, 1), lambda i: (0, 0)),
        ],
        out_specs=pl.BlockSpec((_RBE, 1), lambda i: (i, 0)),
        out_shape=jax.ShapeDtypeStruct((N_EDGES, 1), jnp.float32),
    )(hi, hj, ea, w1aT, w1bT, w1cT, b1.reshape(1, -1), w2, b2.reshape(1, 1))


def kernel(x, edge_index, edge_attr, h,
           W_ih, W_hh, b_ih, b_hh,
           Wl1, bl1, Wr1, Wl2, bl2, Wr2, Wl3, bl3, Wr3,
           Wc1, bc1, Wc2, bc2):
    src = edge_index[0].astype(jnp.int32)
    dst = edge_index[1].astype(jnp.int32)
    pad = NE_PAD - N_EDGES
    src_p = jnp.concatenate([src, jnp.zeros((pad,), jnp.int32)])
    dst_p = jnp.concatenate([dst, jnp.full((pad,), N_NODES, jnp.int32)])
    src2d = src_p.reshape(NW, NCHUNK, CHUNK)
    dst2d = dst_p.reshape(NW, NCHUNK, CHUNK)
    src64 = src_p.reshape(NW, NCH_AG, CH_AG)
    dst64 = dst_p.reshape(NW, NCH_AG, CH_AG)
    pc = _cnt(dst2d)
    h1 = _gru_tc(x, h, W_ih.T, W_hh.T, b_ih, b_hh)
    p1 = _agg(h1, src64, dst64)
    g1 = _sage_tc(p1, pc, h1, Wl1.T, Wr1.T, bl1)
    p2 = _agg(g1, src64, dst64)
    g2 = _sage_tc(p2, pc, g1, Wl2.T, Wr2.T, bl2)
    p3 = _agg(g2, src64, dst64)
    g3 = _sage_tc(p3, pc, g2, Wl3.T, Wr3.T, bl3)
    hi, hj = _gather2(g3, src2d, dst2d)
    out = _cls_tc(hi, hj, edge_attr,
                  Wc1[:, :D].T, Wc1[:, D:2 * D].T, Wc1[:, 2 * D:].T,
                  bc1, Wc2, bc2)
    return (out, g3)


# speedup vs baseline: 2.1858x; 2.1347x over previous
import functools

import jax
import jax.numpy as jnp
from jax import lax
from jax.experimental import pallas as pl
from jax.experimental.pallas import tpu as pltpu
from jax.experimental.pallas import tpu_sc as plsc

N_NODES = 10000
N_EDGES = 320000
D = 128
EA = 16
NC = 2
NS = 16
NW = NC * NS
CHUNK = 128
NCHUNK = 80
SCH = 8
NSUPER = NCHUNK // SCH
EDGES_PER_TILE = NCHUNK * CHUNK
NE_PAD = NW * EDGES_PER_TILE
ACC_ROWS = 10112
ROWS_PER_TILE = ACC_ROWS // NS


def _sc_mesh():
    return plsc.VectorSubcoreMesh(
        core_axis_name="c", subcore_axis_name="s", num_cores=NC, num_subcores=NS
    )


_ZF = ROWS_PER_TILE // CHUNK
_ZR = ROWS_PER_TILE % CHUNK


CH_AG = 64
NCH_AG = EDGES_PER_TILE // CH_AG


def _make_agg():
    out_type = [jax.ShapeDtypeStruct((NC, ACC_ROWS, D), jnp.float32)]
    scratch = [
        pltpu.VMEM((16, CH_AG), jnp.int32),
        pltpu.VMEM((16, CH_AG), jnp.int32),
        pltpu.VMEM((CH_AG, D), jnp.float32),
        pltpu.VMEM((CH_AG, D), jnp.float32),
        pltpu.VMEM((CH_AG, D), jnp.float32),
        pltpu.VMEM((CH_AG, D), jnp.float32),
        pltpu.VMEM_SHARED((ACC_ROWS, D), jnp.float32),
    ] + [pltpu.SemaphoreType.DMA] * 8

    def body(g, src3d, dst3d, pacc,
             sidx_v, didx_v, buf0, buf1, buf2, buf3, acc_sh, *sems):
        sg = sems[:4]
        ss = sems[4:]
        c = lax.axis_index("c")
        s = lax.axis_index("s")
        wid = s * NC + c
        base = s * ROWS_PER_TILE
        bufs = [buf0, buf1, buf2, buf3]
        zbuf = buf0
        zv = jnp.zeros((16,), jnp.float32)

        def fill(i, carry):
            for k in range(D // 16):
                buf0[i, pl.ds(k * 16, 16)] = zv
            return carry

        lax.fori_loop(0, CH_AG, fill, 0)
        for k in range(ROWS_PER_TILE // CH_AG):
            pltpu.sync_copy(zbuf, acc_sh.at[pl.ds(base + k * CH_AG, CH_AG)])
        pltpu.sync_copy(zbuf.at[pl.ds(0, ROWS_PER_TILE % CH_AG)],
                        acc_sh.at[pl.ds(base + (ROWS_PER_TILE // CH_AG) * CH_AG,
                                        ROWS_PER_TILE % CH_AG)])
        plsc.subcore_barrier()

        def group(grp, carry):
            bj = grp * 16
            pltpu.sync_copy(src3d.at[wid, pl.ds(bj, 16)], sidx_v)
            pltpu.sync_copy(dst3d.at[wid, pl.ds(bj, 16)], didx_v)
            dg = [None] * 4
            ds = [None] * 4
            for q in range(16):
                b = q % 4
                if q >= 4:
                    ds[b].wait()
                dg[b] = pltpu.async_copy(g.at[sidx_v.at[q]], bufs[b],
                                         sg[b])
                if q >= 2:
                    qq = q - 2
                    bb = qq % 4
                    dg[bb].wait()
                    ds[bb] = pltpu.async_copy(bufs[bb],
                                              acc_sh.at[didx_v.at[qq]],
                                              ss[bb], add=True)
            for qq in (14, 15):
                bb = qq % 4
                dg[bb].wait()
                ds[bb] = pltpu.async_copy(bufs[bb], acc_sh.at[didx_v.at[qq]],
                                          ss[bb], add=True)
            for bb in range(4):
                ds[bb].wait()
            return carry

        lax.fori_loop(0, NCH_AG // 16, group, 0)
        plsc.subcore_barrier()
        nf = ROWS_PER_TILE // CH_AG
        nr = ROWS_PER_TILE % CH_AG
        for k in range(nf):
            pltpu.sync_copy(acc_sh.at[pl.ds(base + k * CH_AG, CH_AG)], zbuf)
            pltpu.sync_copy(zbuf, pacc.at[c, pl.ds(base + k * CH_AG, CH_AG)])
        pltpu.sync_copy(acc_sh.at[pl.ds(base + nf * CH_AG, nr)],
                        zbuf.at[pl.ds(0, nr)])
        pltpu.sync_copy(zbuf.at[pl.ds(0, nr)],
                        pacc.at[c, pl.ds(base + nf * CH_AG, nr)])

    return pl.kernel(body, out_type=out_type, mesh=_sc_mesh(),
                     scratch_types=scratch)


def _make_cnt():
    out_type = [jax.ShapeDtypeStruct((NC, ACC_ROWS, D), jnp.float32)]
    scratch = [
        pltpu.VMEM((NCHUNK, CHUNK), jnp.int32),
        pltpu.VMEM((CHUNK, D), jnp.float32),
        pltpu.VMEM((CHUNK, D), jnp.float32),
        pltpu.VMEM_SHARED((ACC_ROWS, D), jnp.float32),
    ]

    def body(dst3d, pcnt, dst_v, ones_v, st_v, cnt_sh):
        c = lax.axis_index("c")
        s = lax.axis_index("s")
        wid = s * NC + c
        base = s * ROWS_PER_TILE
        zv = jnp.zeros((16,), jnp.float32)
        ov = jnp.ones((16,), jnp.float32)

        def fill(i, carry):
            for k in range(D // 16):
                ones_v[i, pl.ds(k * 16, 16)] = ov
                st_v[i, pl.ds(k * 16, 16)] = zv
            return carry

        lax.fori_loop(0, CHUNK, fill, 0)
        for k in range(_ZF):
            pltpu.sync_copy(st_v, cnt_sh.at[pl.ds(base + k * CHUNK, CHUNK)])
        pltpu.sync_copy(st_v.at[pl.ds(0, _ZR)],
                        cnt_sh.at[pl.ds(base + _ZF * CHUNK, _ZR)])
        pltpu.sync_copy(dst3d.at[wid], dst_v)
        plsc.subcore_barrier()

        def step(j, carry):
            pltpu.sync_copy(ones_v, cnt_sh.at[dst_v.at[j]], add=True)
            return carry

        lax.fori_loop(0, NCHUNK, step, 0)
        plsc.subcore_barrier()
        for k in range(_ZF):
            pltpu.sync_copy(cnt_sh.at[pl.ds(base + k * CHUNK, CHUNK)], st_v)
            pltpu.sync_copy(st_v, pcnt.at[c, pl.ds(base + k * CHUNK, CHUNK)])
        pltpu.sync_copy(cnt_sh.at[pl.ds(base + _ZF * CHUNK, _ZR)],
                        st_v.at[pl.ds(0, _ZR)])
        pltpu.sync_copy(st_v.at[pl.ds(0, _ZR)],
                        pcnt.at[c, pl.ds(base + _ZF * CHUNK, _ZR)])

    return pl.kernel(body, out_type=out_type, mesh=_sc_mesh(),
                     scratch_types=scratch)


@functools.lru_cache(maxsize=None)
def _get_agg():
    return _make_agg()


@functools.lru_cache(maxsize=None)
def _get_cnt():
    return _make_cnt()


def _cnt(dst3d):
    (pc,) = _get_cnt()(dst3d)
    return pc


def _agg(g, src2d, dst2d):
    (p,) = _get_agg()(g, src2d, dst2d)
    return p


def _make_gather2():
    out_type = [
        jax.ShapeDtypeStruct((NE_PAD, D), jnp.float32),
        jax.ShapeDtypeStruct((NE_PAD, D), jnp.float32),
    ]
    scratch = [
        pltpu.VMEM((NCHUNK, CHUNK), jnp.int32),
        pltpu.VMEM((CHUNK, D), jnp.float32),
        pltpu.VMEM((CHUNK, D), jnp.float32),
        pltpu.VMEM((CHUNK, D), jnp.float32),
        pltpu.VMEM((CHUNK, D), jnp.float32),
    ] + [pltpu.SemaphoreType.DMA] * 8

    def body(g, src3d, dst3d, hi, hj, idx_v, buf0, buf1, buf2, buf3, *sems):
        sg = sems[:4]
        sw = sems[4:]
        c = lax.axis_index("c")
        s = lax.axis_index("s")
        bufs = [buf0, buf1, buf2, buf3]

        def pipe(idx3d, out):
            for ph in range(2):
                r = 2 * s + ph
                pltpu.sync_copy(idx3d.at[r], idx_v)

                def group(grp, carry):
                    bj = grp * 16
                    base_e = r * EDGES_PER_TILE + bj * CHUNK
                    dg = [None] * 4
                    dw = [None] * 4
                    for q in range(16):
                        b = q % 4
                        if q >= 4:
                            dw[b].wait()
                        dg[b] = pltpu.async_copy(g.at[idx_v.at[bj + q]],
                                                 bufs[b], sg[b])
                        if q >= 2:
                            qq = q - 2
                            bb = qq % 4
                            dg[bb].wait()
                            dw[bb] = pltpu.async_copy(
                                bufs[bb],
                                out.at[pl.ds(base_e + qq * CHUNK, CHUNK)],
                                sw[bb])
                    for qq in (14, 15):
                        bb = qq % 4
                        dg[bb].wait()
                        dw[bb] = pltpu.async_copy(
                            bufs[bb],
                            out.at[pl.ds(base_e + qq * CHUNK, CHUNK)],
                            sw[bb])
                    for bb in range(4):
                        dw[bb].wait()
                    return carry

                lax.fori_loop(0, NCHUNK // 16, group, 0)

        pl.when(c == 0)(lambda: pipe(src3d, hi))
        pl.when(c == 1)(lambda: pipe(dst3d, hj))

    return pl.kernel(body, out_type=out_type, mesh=_sc_mesh(),
                     scratch_types=scratch)


@functools.lru_cache(maxsize=None)
def _get_gather2():
    return _make_gather2()


def _gather2(g, src2d, dst2d):
    return _get_gather2()(g, src2d, dst2d)


_RB = 1000
_RBE = 1280


def _gru_tc(x, h, wihT, whhT, bih, bhh):
    def body(x_r, h_r, wi_r, wh_r, bi_r, bh_r, o_r):
        hb = h_r[...]
        gi = jnp.dot(x_r[...], wi_r[...], preferred_element_type=jnp.float32) + bi_r[...]
        gh = jnp.dot(hb, wh_r[...], preferred_element_type=jnp.float32) + bh_r[...]
        r = jax.nn.sigmoid(gi[:, :D] + gh[:, :D])
        z = jax.nn.sigmoid(gi[:, D:2 * D] + gh[:, D:2 * D])
        n = jnp.tanh(gi[:, 2 * D:] + r * gh[:, 2 * D:])
        o_r[...] = (1.0 - z) * n + z * hb

    return pl.pallas_call(
        body,
        grid=(N_NODES // _RB,),
        in_specs=[
            pl.BlockSpec((_RB, D), lambda i: (i, 0)),
            pl.BlockSpec((_RB, D), lambda i: (i, 0)),
            pl.BlockSpec((D, 3 * D), lambda i: (0, 0)),
            pl.BlockSpec((D, 3 * D), lambda i: (0, 0)),
            pl.BlockSpec((1, 3 * D), lambda i: (0, 0)),
            pl.BlockSpec((1, 3 * D), lambda i: (0, 0)),
        ],
        out_specs=pl.BlockSpec((_RB, D), lambda i: (i, 0)),
        out_shape=jax.ShapeDtypeStruct((N_NODES, D), jnp.float32),
    )(x, h, wihT, whhT, bih.reshape(1, -1), bhh.reshape(1, -1))


def _sage_tc(pacc, pcnt, g, wlT, wrT, bl):
    def body(p_r, c_r, g_r, wl_r, wr_r, bl_r, o_r):
        ssum = p_r[0] + p_r[1]
        cnt = c_r[0] + c_r[1]
        inv = 1.0 / jnp.maximum(cnt[:, 0:1], 1.0)
        mean = ssum * inv
        acc = jnp.dot(mean, wl_r[...], preferred_element_type=jnp.float32)
        acc = acc + jnp.dot(g_r[...], wr_r[...], preferred_element_type=jnp.float32)
        o_r[...] = jnp.maximum(acc + bl_r[...], 0.0)

    return pl.pallas_call(
        body,
        grid=(N_NODES // _RB,),
        in_specs=[
            pl.BlockSpec((NC, _RB, D), lambda i: (0, i, 0)),
            pl.BlockSpec((NC, _RB, D), lambda i: (0, i, 0)),
            pl.BlockSpec((_RB, D), lambda i: (i, 0)),
            pl.BlockSpec((D, D), lambda i: (0, 0)),
            pl.BlockSpec((D, D), lambda i: (0, 0)),
            pl.BlockSpec((1, D), lambda i: (0, 0)),
        ],
        out_specs=pl.BlockSpec((_RB, D), lambda i: (i, 0)),
        out_shape=jax.ShapeDtypeStruct((N_NODES, D), jnp.float32),
    )(pacc, pcnt, g, wlT, wrT, bl.reshape(1, -1))


def _cls_tc(hi, hj, ea, w1aT, w1bT, w1cT, b1, w2, b2):
    def body(hi_r, hj_r, ea_r, wa_r, wb_r, wc_r, b1_r, w2_r, b2_r, o_r):
        hid = jnp.dot(hi_r[...], wa_r[...], preferred_element_type=jnp.float32)
        hid = hid + jnp.dot(hj_r[...], wb_r[...], preferred_element_type=jnp.float32)
        hid = hid + jnp.dot(ea_r[...], wc_r[...], preferred_element_type=jnp.float32)
        hid = jnp.maximum(hid + b1_r[...], 0.0)
        o_r[...] = jnp.sum(hid * w2_r[...], axis=1, keepdims=True) + b2_r[...]

    return pl.pallas_call(
        body,
        grid=(N_EDGES // _RBE,),
        in_specs=[
            pl.BlockSpec((_RBE, D), lambda i: (i, 0)),
            pl.BlockSpec((_RBE, D), lambda i: (i, 0)),
            pl.BlockSpec((_RBE, EA), lambda i: (i, 0)),
            pl.BlockSpec((D, 2 * D), lambda i: (0, 0)),
            pl.BlockSpec((D, 2 * D), lambda i: (0, 0)),
            pl.BlockSpec((EA, 2 * D), lambda i: (0, 0)),
            pl.BlockSpec((1, 2 * D), lambda i: (0, 0)),
            pl.BlockSpec((1, 2 * D), lambda i: (0, 0)),
            pl.BlockSpec((1, 1), lambda i: (0, 0)),
        ],
        out_specs=pl.BlockSpec((_RBE, 1), lambda i: (i, 0)),
        out_shape=jax.ShapeDtypeStruct((N_EDGES, 1), jnp.float32),
    )(hi, hj, ea, w1aT, w1bT, w1cT, b1.reshape(1, -1), w2, b2.reshape(1, 1))


def kernel(x, edge_index, edge_attr, h,
           W_ih, W_hh, b_ih, b_hh,
           Wl1, bl1, Wr1, Wl2, bl2, Wr2, Wl3, bl3, Wr3,
           Wc1, bc1, Wc2, bc2):
    src = edge_index[0].astype(jnp.int32)
    dst = edge_index[1].astype(jnp.int32)
    pad = NE_PAD - N_EDGES
    pad_src = (jnp.arange(pad, dtype=jnp.int32) * 79) % N_NODES
    pad_dst = N_NODES + (jnp.arange(pad, dtype=jnp.int32) % (ACC_ROWS - N_NODES))
    src_p = jnp.concatenate([src, pad_src])
    dst_p = jnp.concatenate([dst, pad_dst])
    dst_pg = jnp.concatenate([dst, pad_src])
    src2d = src_p.reshape(NW, NCHUNK, CHUNK)
    dst2d = dst_pg.reshape(NW, NCHUNK, CHUNK)
    dst2d_s = dst_p.reshape(NW, NCHUNK, CHUNK)
    src64 = src_p.reshape(NW, NCH_AG, CH_AG)
    dst64 = dst_p.reshape(NW, NCH_AG, CH_AG)
    pc = _cnt(dst2d_s)
    h1 = _gru_tc(x, h, W_ih.T, W_hh.T, b_ih, b_hh)
    p1 = _agg(h1, src64, dst64)
    g1 = _sage_tc(p1, pc, h1, Wl1.T, Wr1.T, bl1)
    p2 = _agg(g1, src64, dst64)
    g2 = _sage_tc(p2, pc, g1, Wl2.T, Wr2.T, bl2)
    p3 = _agg(g2, src64, dst64)
    g3 = _sage_tc(p3, pc, g2, Wl3.T, Wr3.T, bl3)
    hi, hj = _gather2(g3, src2d, dst2d)
    out = _cls_tc(hi, hj, edge_attr,
                  Wc1[:, :D].T, Wc1[:, D:2 * D].T, Wc1[:, 2 * D:].T,
                  bc1, Wc2, bc2)
    return (out, g3)


# speedup vs baseline: 2.2001x; 1.0065x over previous
import functools

import jax
import jax.numpy as jnp
from jax import lax
from jax.experimental import pallas as pl
from jax.experimental.pallas import tpu as pltpu
from jax.experimental.pallas import tpu_sc as plsc

N_NODES = 10000
N_EDGES = 320000
D = 128
EA = 16
NC = 2
NS = 16
NW = NC * NS
CHUNK = 128
NCHUNK = 80
SCH = 8
NSUPER = NCHUNK // SCH
EDGES_PER_TILE = NCHUNK * CHUNK
NE_PAD = NW * EDGES_PER_TILE
ACC_ROWS = 10112
ROWS_PER_TILE = ACC_ROWS // NS


def _sc_mesh():
    return plsc.VectorSubcoreMesh(
        core_axis_name="c", subcore_axis_name="s", num_cores=NC, num_subcores=NS
    )


_ZF = ROWS_PER_TILE // CHUNK
_ZR = ROWS_PER_TILE % CHUNK


CH_AG = 64
NCH_AG = EDGES_PER_TILE // CH_AG


def _make_agg():
    out_type = [jax.ShapeDtypeStruct((NC, ACC_ROWS, D), jnp.float32)]
    scratch = [
        pltpu.VMEM((16, CH_AG), jnp.int32),
        pltpu.VMEM((16, CH_AG), jnp.int32),
        pltpu.VMEM((CH_AG, D), jnp.float32),
        pltpu.VMEM((CH_AG, D), jnp.float32),
        pltpu.VMEM((CH_AG, D), jnp.float32),
        pltpu.VMEM((CH_AG, D), jnp.float32),
        pltpu.VMEM_SHARED((ACC_ROWS, D), jnp.float32),
    ] + [pltpu.SemaphoreType.DMA] * 8

    def body(g, src3d, dst3d, pacc,
             sidx_v, didx_v, buf0, buf1, buf2, buf3, acc_sh, *sems):
        sg = sems[:4]
        ss = sems[4:]
        c = lax.axis_index("c")
        s = lax.axis_index("s")
        wid = s * NC + c
        base = s * ROWS_PER_TILE
        bufs = [buf0, buf1, buf2, buf3]
        zbuf = buf0
        zv = jnp.zeros((16,), jnp.float32)

        def fill(i, carry):
            for k in range(D // 16):
                buf0[i, pl.ds(k * 16, 16)] = zv
            return carry

        lax.fori_loop(0, CH_AG, fill, 0)
        for k in range(ROWS_PER_TILE // CH_AG):
            pltpu.sync_copy(zbuf, acc_sh.at[pl.ds(base + k * CH_AG, CH_AG)])
        pltpu.sync_copy(zbuf.at[pl.ds(0, ROWS_PER_TILE % CH_AG)],
                        acc_sh.at[pl.ds(base + (ROWS_PER_TILE // CH_AG) * CH_AG,
                                        ROWS_PER_TILE % CH_AG)])
        plsc.subcore_barrier()

        def group(grp, carry):
            bj = grp * 16
            pltpu.sync_copy(src3d.at[wid, pl.ds(bj, 16)], sidx_v)
            pltpu.sync_copy(dst3d.at[wid, pl.ds(bj, 16)], didx_v)
            dg = [None] * 4
            ds = [None] * 4
            for q in range(16):
                b = q % 4
                if q >= 4:
                    ds[b].wait()
                dg[b] = pltpu.async_copy(g.at[sidx_v.at[q]], bufs[b],
                                         sg[b])
                if q >= 2:
                    qq = q - 2
                    bb = qq % 4
                    dg[bb].wait()
                    ds[bb] = pltpu.async_copy(bufs[bb],
                                              acc_sh.at[didx_v.at[qq]],
                                              ss[bb], add=True)
            for qq in (14, 15):
                bb = qq % 4
                dg[bb].wait()
                ds[bb] = pltpu.async_copy(bufs[bb], acc_sh.at[didx_v.at[qq]],
                                          ss[bb], add=True)
            for bb in range(4):
                ds[bb].wait()
            return carry

        lax.fori_loop(0, NCH_AG // 16, group, 0)
        plsc.subcore_barrier()
        nf = ROWS_PER_TILE // CH_AG
        nr = ROWS_PER_TILE % CH_AG
        for k in range(nf):
            pltpu.sync_copy(acc_sh.at[pl.ds(base + k * CH_AG, CH_AG)], zbuf)
            pltpu.sync_copy(zbuf, pacc.at[c, pl.ds(base + k * CH_AG, CH_AG)])
        pltpu.sync_copy(acc_sh.at[pl.ds(base + nf * CH_AG, nr)],
                        zbuf.at[pl.ds(0, nr)])
        pltpu.sync_copy(zbuf.at[pl.ds(0, nr)],
                        pacc.at[c, pl.ds(base + nf * CH_AG, nr)])

    return pl.kernel(body, out_type=out_type, mesh=_sc_mesh(),
                     scratch_types=scratch)


def _make_cnt():
    out_type = [jax.ShapeDtypeStruct((NC, ACC_ROWS, D), jnp.float32)]
    scratch = [
        pltpu.VMEM((NCHUNK, CHUNK), jnp.int32),
        pltpu.VMEM((CHUNK, D), jnp.float32),
        pltpu.VMEM((CHUNK, D), jnp.float32),
        pltpu.VMEM_SHARED((ACC_ROWS, D), jnp.float32),
    ]

    def body(dst3d, pcnt, dst_v, ones_v, st_v, cnt_sh):
        c = lax.axis_index("c")
        s = lax.axis_index("s")
        wid = s * NC + c
        base = s * ROWS_PER_TILE
        zv = jnp.zeros((16,), jnp.float32)
        ov = jnp.ones((16,), jnp.float32)

        def fill(i, carry):
            for k in range(D // 16):
                ones_v[i, pl.ds(k * 16, 16)] = ov
                st_v[i, pl.ds(k * 16, 16)] = zv
            return carry

        lax.fori_loop(0, CHUNK, fill, 0)
        for k in range(_ZF):
            pltpu.sync_copy(st_v, cnt_sh.at[pl.ds(base + k * CHUNK, CHUNK)])
        pltpu.sync_copy(st_v.at[pl.ds(0, _ZR)],
                        cnt_sh.at[pl.ds(base + _ZF * CHUNK, _ZR)])
        pltpu.sync_copy(dst3d.at[wid], dst_v)
        plsc.subcore_barrier()

        def step(j, carry):
            pltpu.sync_copy(ones_v, cnt_sh.at[dst_v.at[j]], add=True)
            return carry

        lax.fori_loop(0, NCHUNK, step, 0)
        plsc.subcore_barrier()
        for k in range(_ZF):
            pltpu.sync_copy(cnt_sh.at[pl.ds(base + k * CHUNK, CHUNK)], st_v)
            pltpu.sync_copy(st_v, pcnt.at[c, pl.ds(base + k * CHUNK, CHUNK)])
        pltpu.sync_copy(cnt_sh.at[pl.ds(base + _ZF * CHUNK, _ZR)],
                        st_v.at[pl.ds(0, _ZR)])
        pltpu.sync_copy(st_v.at[pl.ds(0, _ZR)],
                        pcnt.at[c, pl.ds(base + _ZF * CHUNK, _ZR)])

    return pl.kernel(body, out_type=out_type, mesh=_sc_mesh(),
                     scratch_types=scratch)


@functools.lru_cache(maxsize=None)
def _get_agg():
    return _make_agg()


@functools.lru_cache(maxsize=None)
def _get_cnt():
    return _make_cnt()


def _cnt(dst3d):
    (pc,) = _get_cnt()(dst3d)
    return pc


def _agg(g, src2d, dst2d):
    (p,) = _get_agg()(g, src2d, dst2d)
    return p


NE_HALF = NE_PAD // 2


def _make_gather2(half: int):
    out_type = [
        jax.ShapeDtypeStruct((NE_HALF, D), jnp.float32),
        jax.ShapeDtypeStruct((NE_HALF, D), jnp.float32),
    ]
    scratch = [
        pltpu.VMEM((NCHUNK, CHUNK), jnp.int32),
        pltpu.VMEM((CHUNK, D), jnp.float32),
        pltpu.VMEM((CHUNK, D), jnp.float32),
        pltpu.VMEM((CHUNK, D), jnp.float32),
        pltpu.VMEM((CHUNK, D), jnp.float32),
    ] + [pltpu.SemaphoreType.DMA] * 8

    def body(g, src3d, dst3d, hi, hj, idx_v, buf0, buf1, buf2, buf3, *sems):
        sg = sems[:4]
        sw = sems[4:]
        c = lax.axis_index("c")
        s = lax.axis_index("s")
        bufs = [buf0, buf1, buf2, buf3]

        def pipe(idx3d, out):
            r = half * NS + s
            pltpu.sync_copy(idx3d.at[r], idx_v)

            def group(grp, carry):
                bj = grp * 16
                base_e = s * EDGES_PER_TILE + bj * CHUNK
                dg = [None] * 4
                dw = [None] * 4
                for q in range(16):
                    b = q % 4
                    if q >= 4:
                        dw[b].wait()
                    dg[b] = pltpu.async_copy(g.at[idx_v.at[bj + q]],
                                             bufs[b], sg[b])
                    if q >= 2:
                        qq = q - 2
                        bb = qq % 4
                        dg[bb].wait()
                        dw[bb] = pltpu.async_copy(
                            bufs[bb],
                            out.at[pl.ds(base_e + qq * CHUNK, CHUNK)],
                            sw[bb])
                for qq in (14, 15):
                    bb = qq % 4
                    dg[bb].wait()
                    dw[bb] = pltpu.async_copy(
                        bufs[bb],
                        out.at[pl.ds(base_e + qq * CHUNK, CHUNK)],
                        sw[bb])
                for bb in range(4):
                    dw[bb].wait()
                return carry

            lax.fori_loop(0, NCHUNK // 16, group, 0)

        pl.when(c == 0)(lambda: pipe(src3d, hi))
        pl.when(c == 1)(lambda: pipe(dst3d, hj))

    return pl.kernel(body, out_type=out_type, mesh=_sc_mesh(),
                     scratch_types=scratch)


@functools.lru_cache(maxsize=None)
def _get_gather2(half: int):
    return _make_gather2(half)


def _gather2_half(half, g, src2d, dst2d):
    return _get_gather2(half)(g, src2d, dst2d)


_RB = 1000
_RBE = 1280


def _gru_tc(x, h, wihT, whhT, bih, bhh):
    def body(x_r, h_r, wi_r, wh_r, bi_r, bh_r, o_r):
        hb = h_r[...]
        gi = jnp.dot(x_r[...], wi_r[...], preferred_element_type=jnp.float32) + bi_r[...]
        gh = jnp.dot(hb, wh_r[...], preferred_element_type=jnp.float32) + bh_r[...]
        r = jax.nn.sigmoid(gi[:, :D] + gh[:, :D])
        z = jax.nn.sigmoid(gi[:, D:2 * D] + gh[:, D:2 * D])
        n = jnp.tanh(gi[:, 2 * D:] + r * gh[:, 2 * D:])
        o_r[...] = (1.0 - z) * n + z * hb

    return pl.pallas_call(
        body,
        grid=(N_NODES // _RB,),
        in_specs=[
            pl.BlockSpec((_RB, D), lambda i: (i, 0)),
            pl.BlockSpec((_RB, D), lambda i: (i, 0)),
            pl.BlockSpec((D, 3 * D), lambda i: (0, 0)),
            pl.BlockSpec((D, 3 * D), lambda i: (0, 0)),
            pl.BlockSpec((1, 3 * D), lambda i: (0, 0)),
            pl.BlockSpec((1, 3 * D), lambda i: (0, 0)),
        ],
        out_specs=pl.BlockSpec((_RB, D), lambda i: (i, 0)),
        out_shape=jax.ShapeDtypeStruct((N_NODES, D), jnp.float32),
    )(x, h, wihT, whhT, bih.reshape(1, -1), bhh.reshape(1, -1))


def _sage_tc(pacc, pcnt, g, wlT, wrT, bl):
    def body(p_r, c_r, g_r, wl_r, wr_r, bl_r, o_r):
        ssum = p_r[0] + p_r[1]
        cnt = c_r[0] + c_r[1]
        inv = 1.0 / jnp.maximum(cnt[:, 0:1], 1.0)
        mean = ssum * inv
        acc = jnp.dot(mean, wl_r[...], preferred_element_type=jnp.float32)
        acc = acc + jnp.dot(g_r[...], wr_r[...], preferred_element_type=jnp.float32)
        o_r[...] = jnp.maximum(acc + bl_r[...], 0.0)

    return pl.pallas_call(
        body,
        grid=(N_NODES // _RB,),
        in_specs=[
            pl.BlockSpec((NC, _RB, D), lambda i: (0, i, 0)),
            pl.BlockSpec((NC, _RB, D), lambda i: (0, i, 0)),
            pl.BlockSpec((_RB, D), lambda i: (i, 0)),
            pl.BlockSpec((D, D), lambda i: (0, 0)),
            pl.BlockSpec((D, D), lambda i: (0, 0)),
            pl.BlockSpec((1, D), lambda i: (0, 0)),
        ],
        out_specs=pl.BlockSpec((_RB, D), lambda i: (i, 0)),
        out_shape=jax.ShapeDtypeStruct((N_NODES, D), jnp.float32),
    )(pacc, pcnt, g, wlT, wrT, bl.reshape(1, -1))


def _cls_tc(hi, hj, ea, w1aT, w1bT, w1cT, b1, w2, b2, nrows):
    def body(hi_r, hj_r, ea_r, wa_r, wb_r, wc_r, b1_r, w2_r, b2_r, o_r):
        hid = jnp.dot(hi_r[...], wa_r[...], preferred_element_type=jnp.float32)
        hid = hid + jnp.dot(hj_r[...], wb_r[...], preferred_element_type=jnp.float32)
        hid = hid + jnp.dot(ea_r[...], wc_r[...], preferred_element_type=jnp.float32)
        hid = jnp.maximum(hid + b1_r[...], 0.0)
        o_r[...] = jnp.sum(hid * w2_r[...], axis=1, keepdims=True) + b2_r[...]

    return pl.pallas_call(
        body,
        grid=(nrows // _RBE,),
        in_specs=[
            pl.BlockSpec((_RBE, D), lambda i: (i, 0)),
            pl.BlockSpec((_RBE, D), lambda i: (i, 0)),
            pl.BlockSpec((_RBE, EA), lambda i: (i, 0)),
            pl.BlockSpec((D, 2 * D), lambda i: (0, 0)),
            pl.BlockSpec((D, 2 * D), lambda i: (0, 0)),
            pl.BlockSpec((EA, 2 * D), lambda i: (0, 0)),
            pl.BlockSpec((1, 2 * D), lambda i: (0, 0)),
            pl.BlockSpec((1, 2 * D), lambda i: (0, 0)),
            pl.BlockSpec((1, 1), lambda i: (0, 0)),
        ],
        out_specs=pl.BlockSpec((_RBE, 1), lambda i: (i, 0)),
        out_shape=jax.ShapeDtypeStruct((nrows, 1), jnp.float32),
    )(hi, hj, ea, w1aT, w1bT, w1cT, b1.reshape(1, -1), w2, b2.reshape(1, 1))


def kernel(x, edge_index, edge_attr, h,
           W_ih, W_hh, b_ih, b_hh,
           Wl1, bl1, Wr1, Wl2, bl2, Wr2, Wl3, bl3, Wr3,
           Wc1, bc1, Wc2, bc2):
    src = edge_index[0].astype(jnp.int32)
    dst = edge_index[1].astype(jnp.int32)
    pad = NE_PAD - N_EDGES
    pad_src = (jnp.arange(pad, dtype=jnp.int32) * 79) % N_NODES
    pad_dst = N_NODES + (jnp.arange(pad, dtype=jnp.int32) % (ACC_ROWS - N_NODES))
    src_p = jnp.concatenate([src, pad_src])
    dst_p = jnp.concatenate([dst, pad_dst])
    dst_pg = jnp.concatenate([dst, pad_src])
    src2d = src_p.reshape(NW, NCHUNK, CHUNK)
    dst2d = dst_pg.reshape(NW, NCHUNK, CHUNK)
    dst2d_s = dst_p.reshape(NW, NCHUNK, CHUNK)
    src64 = src_p.reshape(NW, NCH_AG, CH_AG)
    dst64 = dst_p.reshape(NW, NCH_AG, CH_AG)
    pc = _cnt(dst2d_s)
    h1 = _gru_tc(x, h, W_ih.T, W_hh.T, b_ih, b_hh)
    p1 = _agg(h1, src64, dst64)
    g1 = _sage_tc(p1, pc, h1, Wl1.T, Wr1.T, bl1)
    p2 = _agg(g1, src64, dst64)
    g2 = _sage_tc(p2, pc, g1, Wl2.T, Wr2.T, bl2)
    p3 = _agg(g2, src64, dst64)
    g3 = _sage_tc(p3, pc, g2, Wl3.T, Wr3.T, bl3)
    hi_a, hj_a = _gather2_half(0, g3, src2d, dst2d)
    hi_b, hj_b = _gather2_half(1, g3, src2d, dst2d)
    w1a, w1b, w1c = Wc1[:, :D].T, Wc1[:, D:2 * D].T, Wc1[:, 2 * D:].T
    nb = N_EDGES - NE_HALF
    out_a = _cls_tc(hi_a, hj_a, edge_attr[:NE_HALF],
                    w1a, w1b, w1c, bc1, Wc2, bc2, NE_HALF)
    out_b = _cls_tc(hi_b, hj_b, edge_attr[NE_HALF:],
                    w1a, w1b, w1c, bc1, Wc2, bc2, nb)
    out = jnp.concatenate([out_a, out_b], axis=0)
    return (out, g3)


# speedup vs baseline: 2.2020x; 1.0009x over previous
import functools

import jax
import jax.numpy as jnp
from jax import lax
from jax.experimental import pallas as pl
from jax.experimental.pallas import tpu as pltpu
from jax.experimental.pallas import tpu_sc as plsc

N_NODES = 10000
N_EDGES = 320000
D = 128
EA = 16
NC = 2
NS = 16
NW = NC * NS
CHUNK = 128
NCHUNK = 80
EDGES_PER_TILE = NCHUNK * CHUNK
NE_PAD = NW * EDGES_PER_TILE
ACC_ROWS = 10112
ROWS_PER_TILE = ACC_ROWS // NS


def _sc_mesh():
    return plsc.VectorSubcoreMesh(
        core_axis_name="c", subcore_axis_name="s", num_cores=NC, num_subcores=NS
    )


_ZF = ROWS_PER_TILE // CHUNK
_ZR = ROWS_PER_TILE % CHUNK


CH_AG = 64
NCH_AG = EDGES_PER_TILE // CH_AG


def _make_agg():
    out_type = [jax.ShapeDtypeStruct((NC, ACC_ROWS, D), jnp.float32)]
    scratch = [
        pltpu.VMEM((16, CH_AG), jnp.int32),
        pltpu.VMEM((16, CH_AG), jnp.int32),
        pltpu.VMEM((CH_AG, D), jnp.float32),
        pltpu.VMEM((CH_AG, D), jnp.float32),
        pltpu.VMEM((CH_AG, D), jnp.float32),
        pltpu.VMEM((CH_AG, D), jnp.float32),
        pltpu.VMEM_SHARED((ACC_ROWS, D), jnp.float32),
    ] + [pltpu.SemaphoreType.DMA] * 8

    def body(g, src3d, dst3d, pacc,
             sidx_v, didx_v, buf0, buf1, buf2, buf3, acc_sh, *sems):
        sg = sems[:4]
        ss = sems[4:]
        c = lax.axis_index("c")
        s = lax.axis_index("s")
        wid = s * NC + c
        base = s * ROWS_PER_TILE
        bufs = [buf0, buf1, buf2, buf3]
        zbuf = buf0
        zv = jnp.zeros((16,), jnp.float32)

        def fill(i, carry):
            for k in range(D // 16):
                buf0[i, pl.ds(k * 16, 16)] = zv
            return carry

        lax.fori_loop(0, CH_AG, fill, 0)
        for k in range(ROWS_PER_TILE // CH_AG):
            pltpu.sync_copy(zbuf, acc_sh.at[pl.ds(base + k * CH_AG, CH_AG)])
        pltpu.sync_copy(zbuf.at[pl.ds(0, ROWS_PER_TILE % CH_AG)],
                        acc_sh.at[pl.ds(base + (ROWS_PER_TILE // CH_AG) * CH_AG,
                                        ROWS_PER_TILE % CH_AG)])
        plsc.subcore_barrier()

        def group(grp, carry):
            bj = grp * 16
            pltpu.sync_copy(src3d.at[wid, pl.ds(bj, 16)], sidx_v)
            pltpu.sync_copy(dst3d.at[wid, pl.ds(bj, 16)], didx_v)
            dg = [None] * 4
            ds = [None] * 4
            for q in range(16):
                b = q % 4
                if q >= 4:
                    ds[b].wait()
                dg[b] = pltpu.async_copy(g.at[sidx_v.at[q]], bufs[b],
                                         sg[b])
                if q >= 2:
                    qq = q - 2
                    bb = qq % 4
                    dg[bb].wait()
                    ds[bb] = pltpu.async_copy(bufs[bb],
                                              acc_sh.at[didx_v.at[qq]],
                                              ss[bb], add=True)
            for qq in (14, 15):
                bb = qq % 4
                dg[bb].wait()
                ds[bb] = pltpu.async_copy(bufs[bb], acc_sh.at[didx_v.at[qq]],
                                          ss[bb], add=True)
            for bb in range(4):
                ds[bb].wait()
            return carry

        lax.fori_loop(0, NCH_AG // 16, group, 0)
        plsc.subcore_barrier()
        nf = ROWS_PER_TILE // CH_AG
        nr = ROWS_PER_TILE % CH_AG
        for k in range(nf):
            pltpu.sync_copy(acc_sh.at[pl.ds(base + k * CH_AG, CH_AG)], zbuf)
            pltpu.sync_copy(zbuf, pacc.at[c, pl.ds(base + k * CH_AG, CH_AG)])
        pltpu.sync_copy(acc_sh.at[pl.ds(base + nf * CH_AG, nr)],
                        zbuf.at[pl.ds(0, nr)])
        pltpu.sync_copy(zbuf.at[pl.ds(0, nr)],
                        pacc.at[c, pl.ds(base + nf * CH_AG, nr)])

    return pl.kernel(body, out_type=out_type, mesh=_sc_mesh(),
                     scratch_types=scratch)


def _make_cnt():
    out_type = [jax.ShapeDtypeStruct((NC, ACC_ROWS, D), jnp.float32)]
    scratch = [
        pltpu.VMEM((NCHUNK, CHUNK), jnp.int32),
        pltpu.VMEM((CHUNK, D), jnp.float32),
        pltpu.VMEM((CHUNK, D), jnp.float32),
        pltpu.VMEM_SHARED((ACC_ROWS, D), jnp.float32),
    ]

    def body(dst3d, pcnt, dst_v, ones_v, st_v, cnt_sh):
        c = lax.axis_index("c")
        s = lax.axis_index("s")
        wid = s * NC + c
        base = s * ROWS_PER_TILE
        zv = jnp.zeros((16,), jnp.float32)
        ov = jnp.ones((16,), jnp.float32)

        def fill(i, carry):
            for k in range(D // 16):
                ones_v[i, pl.ds(k * 16, 16)] = ov
                st_v[i, pl.ds(k * 16, 16)] = zv
            return carry

        lax.fori_loop(0, CHUNK, fill, 0)
        for k in range(_ZF):
            pltpu.sync_copy(st_v, cnt_sh.at[pl.ds(base + k * CHUNK, CHUNK)])
        pltpu.sync_copy(st_v.at[pl.ds(0, _ZR)],
                        cnt_sh.at[pl.ds(base + _ZF * CHUNK, _ZR)])
        pltpu.sync_copy(dst3d.at[wid], dst_v)
        plsc.subcore_barrier()

        def step(j, carry):
            pltpu.sync_copy(ones_v, cnt_sh.at[dst_v.at[j]], add=True)
            return carry

        lax.fori_loop(0, NCHUNK, step, 0)
        plsc.subcore_barrier()
        for k in range(_ZF):
            pltpu.sync_copy(cnt_sh.at[pl.ds(base + k * CHUNK, CHUNK)], st_v)
            pltpu.sync_copy(st_v, pcnt.at[c, pl.ds(base + k * CHUNK, CHUNK)])
        pltpu.sync_copy(cnt_sh.at[pl.ds(base + _ZF * CHUNK, _ZR)],
                        st_v.at[pl.ds(0, _ZR)])
        pltpu.sync_copy(st_v.at[pl.ds(0, _ZR)],
                        pcnt.at[c, pl.ds(base + _ZF * CHUNK, _ZR)])

    return pl.kernel(body, out_type=out_type, mesh=_sc_mesh(),
                     scratch_types=scratch)


@functools.lru_cache(maxsize=None)
def _get_agg():
    return _make_agg()


@functools.lru_cache(maxsize=None)
def _get_cnt():
    return _make_cnt()


def _cnt(dst3d):
    (pc,) = _get_cnt()(dst3d)
    return pc


def _agg(g, src2d, dst2d):
    (p,) = _get_agg()(g, src2d, dst2d)
    return p


NE_HALF = NE_PAD // 2


def _make_gather2(half: int):
    out_type = [
        jax.ShapeDtypeStruct((NE_HALF, D), jnp.float32),
        jax.ShapeDtypeStruct((NE_HALF, D), jnp.float32),
    ]
    scratch = [
        pltpu.VMEM((NCHUNK, CHUNK), jnp.int32),
        pltpu.VMEM((CHUNK, D), jnp.float32),
        pltpu.VMEM((CHUNK, D), jnp.float32),
        pltpu.VMEM((CHUNK, D), jnp.float32),
        pltpu.VMEM((CHUNK, D), jnp.float32),
    ] + [pltpu.SemaphoreType.DMA] * 8

    def body(g, src3d, dst3d, hi, hj, idx_v, buf0, buf1, buf2, buf3, *sems):
        sg = sems[:4]
        sw = sems[4:]
        c = lax.axis_index("c")
        s = lax.axis_index("s")
        bufs = [buf0, buf1, buf2, buf3]

        def pipe(idx3d, out):
            r = half * NS + s
            pltpu.sync_copy(idx3d.at[r], idx_v)

            def group(grp, carry):
                bj = grp * 16
                base_e = s * EDGES_PER_TILE + bj * CHUNK
                dg = [None] * 4
                dw = [None] * 4
                for q in range(16):
                    b = q % 4
                    if q >= 4:
                        dw[b].wait()
                    dg[b] = pltpu.async_copy(g.at[idx_v.at[bj + q]],
                                             bufs[b], sg[b])
                    if q >= 2:
                        qq = q - 2
                        bb = qq % 4
                        dg[bb].wait()
                        dw[bb] = pltpu.async_copy(
                            bufs[bb],
                            out.at[pl.ds(base_e + qq * CHUNK, CHUNK)],
                            sw[bb])
                for qq in (14, 15):
                    bb = qq % 4
                    dg[bb].wait()
                    dw[bb] = pltpu.async_copy(
                        bufs[bb],
                        out.at[pl.ds(base_e + qq * CHUNK, CHUNK)],
                        sw[bb])
                for bb in range(4):
                    dw[bb].wait()
                return carry

            lax.fori_loop(0, NCHUNK // 16, group, 0)

        pl.when(c == 0)(lambda: pipe(src3d, hi))
        pl.when(c == 1)(lambda: pipe(dst3d, hj))

    return pl.kernel(body, out_type=out_type, mesh=_sc_mesh(),
                     scratch_types=scratch)


@functools.lru_cache(maxsize=None)
def _get_gather2(half: int):
    return _make_gather2(half)


def _gather2_half(half, g, src2d, dst2d):
    return _get_gather2(half)(g, src2d, dst2d)


_RB = 1000
_RBE = 1280


def _gru_tc(x, h, wihT, whhT, bih, bhh):
    def body(x_r, h_r, wi_r, wh_r, bi_r, bh_r, o_r):
        hb = h_r[...]
        gi = jnp.dot(x_r[...], wi_r[...], preferred_element_type=jnp.float32) + bi_r[...]
        gh = jnp.dot(hb, wh_r[...], preferred_element_type=jnp.float32) + bh_r[...]
        r = jax.nn.sigmoid(gi[:, :D] + gh[:, :D])
        z = jax.nn.sigmoid(gi[:, D:2 * D] + gh[:, D:2 * D])
        n = jnp.tanh(gi[:, 2 * D:] + r * gh[:, 2 * D:])
        o_r[...] = (1.0 - z) * n + z * hb

    return pl.pallas_call(
        body,
        grid=(N_NODES // _RB,),
        in_specs=[
            pl.BlockSpec((_RB, D), lambda i: (i, 0)),
            pl.BlockSpec((_RB, D), lambda i: (i, 0)),
            pl.BlockSpec((D, 3 * D), lambda i: (0, 0)),
            pl.BlockSpec((D, 3 * D), lambda i: (0, 0)),
            pl.BlockSpec((1, 3 * D), lambda i: (0, 0)),
            pl.BlockSpec((1, 3 * D), lambda i: (0, 0)),
        ],
        out_specs=pl.BlockSpec((_RB, D), lambda i: (i, 0)),
        out_shape=jax.ShapeDtypeStruct((N_NODES, D), jnp.float32),
    )(x, h, wihT, whhT, bih.reshape(1, -1), bhh.reshape(1, -1))


def _sage_tc(pacc, pcnt, g, wlT, wrT, bl):
    def body(p_r, c_r, g_r, wl_r, wr_r, bl_r, o_r):
        ssum = p_r[0] + p_r[1]
        cnt = c_r[0] + c_r[1]
        inv = 1.0 / jnp.maximum(cnt[:, 0:1], 1.0)
        mean = ssum * inv
        acc = jnp.dot(mean, wl_r[...], preferred_element_type=jnp.float32)
        acc = acc + jnp.dot(g_r[...], wr_r[...], preferred_element_type=jnp.float32)
        o_r[...] = jnp.maximum(acc + bl_r[...], 0.0)

    return pl.pallas_call(
        body,
        grid=(N_NODES // _RB,),
        in_specs=[
            pl.BlockSpec((NC, _RB, D), lambda i: (0, i, 0)),
            pl.BlockSpec((NC, _RB, D), lambda i: (0, i, 0)),
            pl.BlockSpec((_RB, D), lambda i: (i, 0)),
            pl.BlockSpec((D, D), lambda i: (0, 0)),
            pl.BlockSpec((D, D), lambda i: (0, 0)),
            pl.BlockSpec((1, D), lambda i: (0, 0)),
        ],
        out_specs=pl.BlockSpec((_RB, D), lambda i: (i, 0)),
        out_shape=jax.ShapeDtypeStruct((N_NODES, D), jnp.float32),
    )(pacc, pcnt, g, wlT, wrT, bl.reshape(1, -1))


def _cls_tc(hi, hj, ea, w1aT, w1bT, w1cT, b1, w2, b2, nrows):
    def body(hi_r, hj_r, ea_r, wa_r, wb_r, wc_r, b1_r, w2_r, b2_r, o_r):
        hid = jnp.dot(hi_r[...], wa_r[...], preferred_element_type=jnp.float32)
        hid = hid + jnp.dot(hj_r[...], wb_r[...], preferred_element_type=jnp.float32)
        hid = hid + jnp.dot(ea_r[...], wc_r[...], preferred_element_type=jnp.float32)
        hid = jnp.maximum(hid + b1_r[...], 0.0)
        o_r[...] = jnp.sum(hid * w2_r[...], axis=1, keepdims=True) + b2_r[...]

    return pl.pallas_call(
        body,
        grid=(nrows // _RBE,),
        in_specs=[
            pl.BlockSpec((_RBE, D), lambda i: (i, 0)),
            pl.BlockSpec((_RBE, D), lambda i: (i, 0)),
            pl.BlockSpec((_RBE, EA), lambda i: (i, 0)),
            pl.BlockSpec((D, 2 * D), lambda i: (0, 0)),
            pl.BlockSpec((D, 2 * D), lambda i: (0, 0)),
            pl.BlockSpec((EA, 2 * D), lambda i: (0, 0)),
            pl.BlockSpec((1, 2 * D), lambda i: (0, 0)),
            pl.BlockSpec((1, 2 * D), lambda i: (0, 0)),
            pl.BlockSpec((1, 1), lambda i: (0, 0)),
        ],
        out_specs=pl.BlockSpec((_RBE, 1), lambda i: (i, 0)),
        out_shape=jax.ShapeDtypeStruct((nrows, 1), jnp.float32),
    )(hi, hj, ea, w1aT, w1bT, w1cT, b1.reshape(1, -1), w2, b2.reshape(1, 1))


def kernel(x, edge_index, edge_attr, h,
           W_ih, W_hh, b_ih, b_hh,
           Wl1, bl1, Wr1, Wl2, bl2, Wr2, Wl3, bl3, Wr3,
           Wc1, bc1, Wc2, bc2):
    src = edge_index[0].astype(jnp.int32)
    dst = edge_index[1].astype(jnp.int32)
    pad = NE_PAD - N_EDGES
    pad_src = (jnp.arange(pad, dtype=jnp.int32) * 79) % N_NODES
    pad_dst = N_NODES + (jnp.arange(pad, dtype=jnp.int32) % (ACC_ROWS - N_NODES))
    src_p = jnp.concatenate([src, pad_src])
    dst_p = jnp.concatenate([dst, pad_dst])
    dst_pg = jnp.concatenate([dst, pad_src])
    src2d = src_p.reshape(NW, NCHUNK, CHUNK)
    dst2d = dst_pg.reshape(NW, NCHUNK, CHUNK)
    dst2d_s = dst_p.reshape(NW, NCHUNK, CHUNK)
    src64 = src_p.reshape(NW, NCH_AG, CH_AG)
    dst64 = dst_p.reshape(NW, NCH_AG, CH_AG)
    pc = _cnt(dst2d_s)
    h1 = _gru_tc(x, h, W_ih.T, W_hh.T, b_ih, b_hh)
    p1 = _agg(h1, src64, dst64)
    g1 = _sage_tc(p1, pc, h1, Wl1.T, Wr1.T, bl1)
    p2 = _agg(g1, src64, dst64)
    g2 = _sage_tc(p2, pc, g1, Wl2.T, Wr2.T, bl2)
    p3 = _agg(g2, src64, dst64)
    g3 = _sage_tc(p3, pc, g2, Wl3.T, Wr3.T, bl3)
    hi_a, hj_a = _gather2_half(0, g3, src2d, dst2d)
    hi_b, hj_b = _gather2_half(1, g3, src2d, dst2d)
    w1a, w1b, w1c = Wc1[:, :D].T, Wc1[:, D:2 * D].T, Wc1[:, 2 * D:].T
    nb = N_EDGES - NE_HALF
    out_a = _cls_tc(hi_a, hj_a, edge_attr[:NE_HALF],
                    w1a, w1b, w1c, bc1, Wc2, bc2, NE_HALF)
    out_b = _cls_tc(hi_b, hj_b, edge_attr[NE_HALF:],
                    w1a, w1b, w1c, bc1, Wc2, bc2, nb)
    out = jnp.concatenate([out_a, out_b], axis=0)
    return (out, g3)
